# baseline (speedup 1.0000x reference)
import functools

import jax
import jax.numpy as jnp
from jax import lax
from jax.experimental import pallas as pl
from jax.experimental.pallas import tpu as pltpu
from jax.experimental.pallas import tpu_sc as plsc

ROWS = 16384
COLS = 2048
NMASK = 128
LANES = 16
NCORES = 2
NSUB = 16
NWORKERS = NCORES * NSUB
ROWS_PER_W = ROWS // NWORKERS
CHUNK = 16
NCHUNK = ROWS_PER_W // CHUNK
CHUNK_ELEMS = CHUNK * COLS


def _sc_body(x_hbm, mask_hbm, out_hbm, mask_v, buf0, buf1,
             lsem0, lsem1, ssem0, ssem1):
    wid = lax.axis_index("s") * NCORES + lax.axis_index("c")
    base = wid * ROWS_PER_W * COLS

    pltpu.sync_copy(mask_hbm, mask_v)

    bufs = (buf0, buf1)
    lsems = (lsem0, lsem1)
    ssems = (ssem0, ssem1)

    def start_load(c, b):
        pltpu.async_copy(x_hbm.at[pl.ds(base + c * CHUNK_ELEMS, CHUNK_ELEMS)],
                         bufs[b], lsems[b])

    def wait_load(b):
        pltpu.make_async_copy(x_hbm.at[pl.ds(base, CHUNK_ELEMS)],
                              bufs[b], lsems[b]).wait()

    def start_store(c, b):
        pltpu.async_copy(bufs[b],
                         out_hbm.at[pl.ds(base + c * CHUNK_ELEMS, CHUNK_ELEMS)],
                         ssems[b])

    def wait_store(b):
        pltpu.make_async_copy(bufs[b],
                              out_hbm.at[pl.ds(base, CHUNK_ELEMS)],
                              ssems[b]).wait()

    def zero_cols(b):
        buf = bufs[b]
        zeros = jnp.zeros((LANES,), jnp.float32)
        for g in range(NMASK // LANES):
            col = mask_v[pl.ds(g * LANES, LANES)]
            for r in range(CHUNK):
                plsc.store_scatter(buf, [col + r * COLS], zeros)

    start_load(0, 0)

    def body(g, carry):
        for b in range(2):
            c = 2 * g + b
            ob = 1 - b
            if b == 0:
                @pl.when(g > 0)
                def _():
                    wait_store(ob)
                start_load(c + 1, ob)
            else:
                wait_store(ob)

                @pl.when(g < NCHUNK // 2 - 1)
                def _():
                    start_load(c + 1, ob)
            wait_load(b)
            zero_cols(b)
            start_store(c, b)
        return carry

    lax.fori_loop(0, NCHUNK // 2, body, 0)
    wait_store(1)


@jax.jit
def kernel(x, mask):
    mesh = plsc.VectorSubcoreMesh(core_axis_name="c", subcore_axis_name="s")
    k = functools.partial(
        pl.kernel,
        mesh=mesh,
        compiler_params=pltpu.CompilerParams(needs_layout_passes=False),
        out_type=jax.ShapeDtypeStruct((ROWS * COLS,), jnp.float32),
        scratch_types=[
            pltpu.VMEM((NMASK,), jnp.int32),
            pltpu.VMEM((CHUNK_ELEMS,), jnp.float32),
            pltpu.VMEM((CHUNK_ELEMS,), jnp.float32),
            pltpu.SemaphoreType.DMA,
            pltpu.SemaphoreType.DMA,
            pltpu.SemaphoreType.DMA,
            pltpu.SemaphoreType.DMA,
        ],
    )(_sc_body)
    return k(x.reshape(ROWS * COLS), mask).reshape(ROWS, COLS)
